# baseline (speedup 1.0000x reference)
import functools

import jax
import jax.numpy as jnp
from jax import lax
from jax.experimental import pallas as pl
from jax.experimental.pallas import tpu as pltpu
from jax.experimental.pallas import tpu_sc as plsc

BATCH = 128
VOCAB = 100000
NC = 2
NS = 16
L = 16
NW = NC * NS
ROWS_PER = BATCH // NW
CHUNK = 20000
NCH = VOCAB // CHUNK
VECS = CHUNK // L
_GATHER_DNUMS = lax.GatherDimensionNumbers(
    offset_dims=(), collapsed_slice_dims=(0,), start_index_map=(0,))


def _shuf(x, perm):
    return lax.gather(
        x, perm[:, None], _GATHER_DNUMS, (1,),
        mode=lax.GatherScatterMode.PROMISE_IN_BOUNDS)


_mesh = plsc.VectorSubcoreMesh(core_axis_name="c", subcore_axis_name="s")


@functools.partial(
    pl.kernel,
    out_type=jax.ShapeDtypeStruct((NW * L,), jnp.int32),
    mesh=_mesh,
    scratch_types=[
        pltpu.VMEM((CHUNK,), jnp.float32),
        pltpu.VMEM((CHUNK,), jnp.float32),
        pltpu.VMEM((L,), jnp.int32),
        pltpu.SemaphoreType.DMA,
        pltpu.SemaphoreType.DMA,
    ],
)
def _argmax_sc(logits_hbm, out_hbm, buf0, buf1, res_ref, sem0, sem1):
    wid = lax.axis_index("s") * NC + lax.axis_index("c")
    row0 = wid * ROWS_PER
    bufs = (buf0, buf1)
    sems = (sem0, sem1)
    tasks = [(r, c) for r in range(ROWS_PER) for c in range(NCH)]
    nt = len(tasks)

    def issue(t):
        r, c = tasks[t]
        return pltpu.async_copy(
            logits_hbm.at[pl.ds((row0 + r) * VOCAB + c * CHUNK, CHUNK)],
            bufs[t % 2],
            sems[t % 2],
        )

    lane = lax.iota(jnp.int32, L)
    res = jnp.zeros((L,), jnp.int32)
    best = bestpos = posvec = None

    handles = [None] * nt
    handles[0] = issue(0)
    for t, (r, c) in enumerate(tasks):
        if t + 1 < nt:
            handles[t + 1] = issue(t + 1)
        handles[t].wait()
        if c == 0:
            best = jnp.full((L,), -jnp.inf, jnp.float32)
            bestpos = lane
            posvec = lane
        buf = bufs[t % 2]

        def body(i, carry, buf=buf):
            bv, bp, pv = carry
            v = buf[pl.ds(i * L, L)]
            m = v > bv
            bv = jnp.where(m, v, bv)
            bp = jnp.where(m, pv, bp)
            return bv, bp, pv + L

        best, bestpos, posvec = lax.fori_loop(
            0, VECS, body, (best, bestpos, posvec)
        )
        if c == NCH - 1:
            bv, bp = best, bestpos
            for s in (8, 4, 2, 1):
                perm = lane ^ s
                ov = _shuf(bv, perm)
                op = _shuf(bp, perm)
                take = (ov > bv) | ((ov == bv) & (op < bp))
                bv = jnp.where(take, ov, bv)
                bp = jnp.where(take, op, bp)
            res = jnp.where(lane == r, bp, res)

    res_ref[...] = res
    pltpu.sync_copy(res_ref, out_hbm.at[pl.ds(wid * L, L)])


def kernel(logits):
    out = _argmax_sc(logits.reshape(BATCH * VOCAB))
    return out.reshape(NW, L)[:, :ROWS_PER].reshape(BATCH)


# speedup vs baseline: 1.4887x; 1.4887x over previous
import functools

import jax
import jax.numpy as jnp
from jax import lax
from jax.experimental import pallas as pl
from jax.experimental.pallas import tpu as pltpu
from jax.experimental.pallas import tpu_sc as plsc

BATCH = 128
VOCAB = 100000
NC = 2
NS = 16
L = 16
NW = NC * NS
ROWS_PER = BATCH // NW
CHUNK = 20000
NCH = VOCAB // CHUNK
VECS = CHUNK // L
U = 10
assert VECS % U == 0
_GATHER_DNUMS = lax.GatherDimensionNumbers(
    offset_dims=(), collapsed_slice_dims=(0,), start_index_map=(0,))


def _shuf(x, perm):
    return lax.gather(
        x, perm[:, None], _GATHER_DNUMS, (1,),
        mode=lax.GatherScatterMode.PROMISE_IN_BOUNDS)


_mesh = plsc.VectorSubcoreMesh(core_axis_name="c", subcore_axis_name="s")


@functools.partial(
    pl.kernel,
    out_type=jax.ShapeDtypeStruct((NW * L,), jnp.int32),
    mesh=_mesh,
    scratch_types=[
        pltpu.VMEM((CHUNK,), jnp.float32),
        pltpu.VMEM((CHUNK,), jnp.float32),
        pltpu.VMEM((L,), jnp.int32),
        pltpu.SemaphoreType.DMA,
        pltpu.SemaphoreType.DMA,
    ],
)
def _argmax_sc(logits_hbm, out_hbm, buf0, buf1, res_ref, sem0, sem1):
    wid = lax.axis_index("s") * NC + lax.axis_index("c")
    row0 = wid * ROWS_PER
    bufs = (buf0, buf1)
    sems = (sem0, sem1)
    tasks = [(r, c) for r in range(ROWS_PER) for c in range(NCH)]
    nt = len(tasks)

    def issue(t):
        r, c = tasks[t]
        return pltpu.async_copy(
            logits_hbm.at[pl.ds((row0 + r) * VOCAB + c * CHUNK, CHUNK)],
            bufs[t % 2],
            sems[t % 2],
        )

    lane = lax.iota(jnp.int32, L)
    res = jnp.zeros((L,), jnp.int32)
    bests = poss = posvec = None

    handles = [None] * nt
    handles[0] = issue(0)
    for t, (r, c) in enumerate(tasks):
        if t + 1 < nt:
            handles[t + 1] = issue(t + 1)
        handles[t].wait()
        if c == 0:
            bests = [jnp.full((L,), -jnp.inf, jnp.float32)] * U
            poss = [lane] * U
            posvec = lane
        buf = bufs[t % 2]

        def body(i, carry, buf=buf):
            bvs, bps, pv = carry
            base = i * (U * L)
            nbvs, nbps = [], []
            for u in range(U):
                v = buf[pl.ds(base + u * L, L)]
                m = v > bvs[u]
                nbvs.append(jnp.where(m, v, bvs[u]))
                nbps.append(jnp.where(m, pv, bps[u]))
            return nbvs, nbps, pv + U * L

        bests, poss, posvec = lax.fori_loop(
            0, VECS // U, body, (bests, poss, posvec)
        )
        if c == NCH - 1:
            bv = bests[0]
            bp = poss[0]
            for u in range(1, U):
                ov = bests[u]
                op = poss[u] + u * L
                take = (ov > bv) | ((ov == bv) & (op < bp))
                bv = jnp.where(take, ov, bv)
                bp = jnp.where(take, op, bp)
            for s in (8, 4, 2, 1):
                perm = lane ^ s
                ov = _shuf(bv, perm)
                op = _shuf(bp, perm)
                take = (ov > bv) | ((ov == bv) & (op < bp))
                bv = jnp.where(take, ov, bv)
                bp = jnp.where(take, op, bp)
            res = jnp.where(lane == r, bp, res)

    res_ref[...] = res
    pltpu.sync_copy(res_ref, out_hbm.at[pl.ds(wid * L, L)])


def kernel(logits):
    out = _argmax_sc(logits.reshape(BATCH * VOCAB))
    return out.reshape(NW, L)[:, :ROWS_PER].reshape(BATCH)
